# baseline (speedup 1.0000x reference)
import functools

import jax
import jax.numpy as jnp
from jax.experimental import pallas as pl
from jax.experimental.pallas import tpu as pltpu

H = 128
B = 2
VSC = 256
VST = 256
BI1 = 128
NI1 = VSC // BI1
BI2 = 64
NI2 = VSC // BI2
N_EDGE = B * VSC * VST
EPS = 1e-5


def _pass1_body(e_ref, h1_ref, h2_ref,
                aw_ref, ab_ref, bw_ref, bb_ref, cw_ref, cb_ref,
                esum_ref, esumsq_ref, ahf_ref, bhf_ref):
    b = pl.program_id(0)
    i = pl.program_id(1)

    @pl.when(jnp.logical_and(b == 0, i == 0))
    def _():
        h1f = h1_ref[...].reshape(B * VSC, H)
        h2f = h2_ref[...].reshape(B * VST, H)
        bias = ab_ref[...] + bb_ref[...] + cb_ref[...]
        ahf_ref[...] = (jnp.dot(h1f, aw_ref[...].T,
                                preferred_element_type=jnp.float32)
                        + bias).reshape(B, VSC, H)
        bhf_ref[...] = jnp.dot(h2f, bw_ref[...].T,
                               preferred_element_type=jnp.float32
                               ).reshape(B, VST, H)

    x = e_ref[0].reshape(BI1 * VST, H)
    ce = jnp.dot(x, cw_ref[...].T, preferred_element_type=jnp.float32)
    ah = ahf_ref[b, pl.ds(i * BI1, BI1), :]
    bh = bhf_ref[b]
    en = ce.reshape(BI1, VST, H) + ah[:, None, :] + bh[None, :, :]

    en2 = en.reshape(BI1 * VST, H)
    psum = jnp.sum(en2, axis=0, keepdims=True)
    psumsq = jnp.sum(en2 * en2, axis=0, keepdims=True)

    @pl.when(jnp.logical_and(b == 0, i == 0))
    def _():
        esum_ref[...] = psum
        esumsq_ref[...] = psumsq

    @pl.when(jnp.logical_or(b != 0, i != 0))
    def _():
        esum_ref[...] += psum
        esumsq_ref[...] += psumsq


def _bn_relu_res(x2d, w, b, res2d):
    m = jnp.mean(x2d, axis=0, keepdims=True)
    v = jnp.mean(x2d * x2d, axis=0, keepdims=True) - m * m
    y = (x2d - m) * jax.lax.rsqrt(v + EPS) * w + b
    return res2d + jnp.maximum(y, 0.0)


def _pass2_body(e_ref, ahf_ref, bhf_ref, cw_ref, esum_ref, esumsq_ref,
                h1_ref, h2_ref,
                u1w_ref, u1b_ref, v1w_ref, v1b_ref,
                u2w_ref, u2b_ref, v2w_ref, v2b_ref,
                nhw_ref, nhb_ref, new_ref, neb_ref,
                eout_ref, h1out_ref, h2out_ref,
                uh1_s, uh2_s, vh1_s, vh2_s, h1agg_s, h2agg_s):
    b = pl.program_id(0)
    i = pl.program_id(1)

    @pl.when(jnp.logical_and(b == 0, i == 0))
    def _():
        h1f = h1_ref[...].reshape(B * VSC, H)
        h2f = h2_ref[...].reshape(B * VST, H)
        uh1_s[...] = (jnp.dot(h1f, u1w_ref[...].T,
                              preferred_element_type=jnp.float32)
                      + u1b_ref[...]).reshape(B, VSC, H)
        uh2_s[...] = (jnp.dot(h2f, u2w_ref[...].T,
                              preferred_element_type=jnp.float32)
                      + u2b_ref[...]).reshape(B, VST, H)
        vh1_s[...] = (jnp.dot(h1f, v1w_ref[...].T,
                              preferred_element_type=jnp.float32)
                      + v1b_ref[...]).reshape(B, VSC, H).astype(jnp.bfloat16)
        vh2_s[...] = (jnp.dot(h2f, v2w_ref[...].T,
                              preferred_element_type=jnp.float32)
                      + v2b_ref[...]).reshape(B, VST, H).astype(jnp.bfloat16)

    mean = esum_ref[...] * (1.0 / N_EDGE)
    var = esumsq_ref[...] * (1.0 / N_EDGE) - mean * mean
    scale = jax.lax.rsqrt(var + EPS) * new_ref[...]
    shift = neb_ref[...] - mean * scale

    x = e_ref[0].reshape(BI2 * VST, H)
    ce = jnp.dot(x, cw_ref[...].T, preferred_element_type=jnp.float32)
    ah = ahf_ref[b, pl.ds(i * BI2, BI2), :]
    bh = bhf_ref[b]
    en = ce.reshape(BI2, VST, H) + ah[:, None, :] + bh[None, :, :]

    y = en.reshape(BI2 * VST, H) * scale + shift
    eout_ref[0] = e_ref[0] + jnp.maximum(y, 0.0).reshape(BI2, VST, H)

    g = jax.nn.sigmoid(en.astype(jnp.bfloat16))
    h1agg_s[b, pl.ds(i * BI2, BI2), :] = jnp.sum(
        g * vh2_s[b][None, :, :], axis=1).astype(jnp.float32)
    part2 = jnp.sum(g * vh1_s[b, pl.ds(i * BI2, BI2), :][:, None, :],
                    axis=0).astype(jnp.float32)

    @pl.when(i == 0)
    def _():
        h2agg_s[b] = part2

    @pl.when(i != 0)
    def _():
        h2agg_s[b] += part2

    @pl.when(jnp.logical_and(b == B - 1, i == NI2 - 1))
    def _():
        x1 = (uh1_s[...] + h1agg_s[...]).reshape(B * VSC, H)
        h1out_ref[...] = _bn_relu_res(
            x1, nhw_ref[...], nhb_ref[...],
            h1_ref[...].reshape(B * VSC, H)).reshape(B, VSC, H)
        x2 = (uh2_s[...] + h2agg_s[...]).reshape(B * VST, H)
        h2out_ref[...] = _bn_relu_res(
            x2, nhw_ref[...], nhb_ref[...],
            h2_ref[...].reshape(B * VST, H)).reshape(B, VST, H)


@functools.partial(jax.jit, static_argnames=())
def kernel(h1, h2, e, graph, U1_w, U1_b, V1_w, V1_b, U2_w, U2_b, V2_w, V2_b,
           A_w, A_b, B_w, B_b, C_w, C_b, nh_w, nh_b, ne_w, ne_b):
    del graph
    r = lambda v: v.reshape(1, H)

    full3 = lambda shape: pl.BlockSpec(shape, lambda b, i: (0, 0, 0))
    full2 = lambda shape: pl.BlockSpec(shape, lambda b, i: (0, 0))
    eblk1 = pl.BlockSpec((1, BI1, VST, H), lambda b, i: (b, i, 0, 0))
    eblk2 = pl.BlockSpec((1, BI2, VST, H), lambda b, i: (b, i, 0, 0))

    f32 = jnp.float32
    wspec = full2((H, H))
    bspec = full2((1, H))

    p1_out_shapes = (
        jax.ShapeDtypeStruct((1, H), f32),
        jax.ShapeDtypeStruct((1, H), f32),
        jax.ShapeDtypeStruct((B, VSC, H), f32),
        jax.ShapeDtypeStruct((B, VST, H), f32),
    )
    p1_out_specs = (
        bspec, bspec, full3((B, VSC, H)), full3((B, VST, H)),
    )
    p1_in_specs = (
        eblk1, full3((B, VSC, H)), full3((B, VST, H)),
        wspec, bspec, wspec, bspec, wspec, bspec,
    )
    esum, esumsq, ahf, bhf = pl.pallas_call(
        _pass1_body,
        grid=(B, NI1),
        in_specs=p1_in_specs,
        out_specs=p1_out_specs,
        out_shape=p1_out_shapes,
    )(e, h1, h2, A_w, r(A_b), B_w, r(B_b), C_w, r(C_b))

    p2_out_shapes = (
        jax.ShapeDtypeStruct((B, VSC, VST, H), f32),
        jax.ShapeDtypeStruct((B, VSC, H), f32),
        jax.ShapeDtypeStruct((B, VST, H), f32),
    )
    p2_out_specs = (eblk2, full3((B, VSC, H)), full3((B, VST, H)))
    p2_in_specs = (
        eblk2, full3((B, VSC, H)), full3((B, VST, H)), wspec,
        bspec, bspec,
        full3((B, VSC, H)), full3((B, VST, H)),
        wspec, bspec, wspec, bspec, wspec, bspec, wspec, bspec,
        bspec, bspec, bspec, bspec,
    )
    e_out, h1_out, h2_out = pl.pallas_call(
        _pass2_body,
        grid=(B, NI2),
        in_specs=p2_in_specs,
        out_specs=p2_out_specs,
        out_shape=p2_out_shapes,
        scratch_shapes=[
            pltpu.VMEM((B, VSC, H), f32),
            pltpu.VMEM((B, VST, H), f32),
            pltpu.VMEM((B, VSC, H), jnp.bfloat16),
            pltpu.VMEM((B, VST, H), jnp.bfloat16),
            pltpu.VMEM((B, VSC, H), f32),
            pltpu.VMEM((B, VST, H), f32),
        ],
    )(e, ahf, bhf, C_w, esum, esumsq, h1, h2,
      U1_w, r(U1_b), V1_w, r(V1_b), U2_w, r(U2_b), V2_w, r(V2_b),
      r(nh_w), r(nh_b), r(ne_w), r(ne_b))

    return (h1_out, h2_out, e_out)


# speedup vs baseline: 1.0085x; 1.0085x over previous
import functools

import jax
import jax.numpy as jnp
from jax.experimental import pallas as pl
from jax.experimental.pallas import tpu as pltpu

H = 128
B = 2
VSC = 256
VST = 256
BI1 = 128
NI1 = VSC // BI1
BI2 = 64
NI2 = VSC // BI2
N_EDGE = B * VSC * VST
EPS = 1e-5


def _pass1_body(e_ref, h1_ref, h2_ref,
                aw_ref, ab_ref, bw_ref, bb_ref, cw_ref, cb_ref,
                esum_ref, esumsq_ref, ahf_ref, bhf_ref):
    b = pl.program_id(0)
    i = pl.program_id(1)

    @pl.when(jnp.logical_and(b == 0, i == 0))
    def _():
        h1f = h1_ref[...].reshape(B * VSC, H)
        h2f = h2_ref[...].reshape(B * VST, H)
        bias = ab_ref[...] + bb_ref[...] + cb_ref[...]
        ahf_ref[...] = (jnp.dot(h1f, aw_ref[...].T,
                                preferred_element_type=jnp.float32)
                        + bias).reshape(B, VSC, H)
        bhf_ref[...] = jnp.dot(h2f, bw_ref[...].T,
                               preferred_element_type=jnp.float32
                               ).reshape(B, VST, H)

    x = e_ref[0].reshape(BI1 * VST, H)
    ce = jnp.dot(x, cw_ref[...].T, preferred_element_type=jnp.float32)
    ah = ahf_ref[b, pl.ds(i * BI1, BI1), :]
    bh = bhf_ref[b]
    en = ce.reshape(BI1, VST, H) + ah[:, None, :] + bh[None, :, :]

    en2 = en.reshape(BI1 * VST, H)
    psum = jnp.sum(en2, axis=0, keepdims=True)
    psumsq = jnp.sum(en2 * en2, axis=0, keepdims=True)

    @pl.when(jnp.logical_and(b == 0, i == 0))
    def _():
        esum_ref[...] = psum
        esumsq_ref[...] = psumsq

    @pl.when(jnp.logical_or(b != 0, i != 0))
    def _():
        esum_ref[...] += psum
        esumsq_ref[...] += psumsq


def _bn_relu_res(x2d, w, b, res2d):
    m = jnp.mean(x2d, axis=0, keepdims=True)
    v = jnp.mean(x2d * x2d, axis=0, keepdims=True) - m * m
    y = (x2d - m) * jax.lax.rsqrt(v + EPS) * w + b
    return res2d + jnp.maximum(y, 0.0)


def _pass2_body(e_ref, ahf_ref, bhf_ref, cw_ref, esum_ref, esumsq_ref,
                h1_ref, h2_ref,
                u1w_ref, u1b_ref, v1w_ref, v1b_ref,
                u2w_ref, u2b_ref, v2w_ref, v2b_ref,
                nhw_ref, nhb_ref, new_ref, neb_ref,
                eout_ref, h1out_ref, h2out_ref,
                uh1_s, uh2_s, vh1_s, vh2_s, h1agg_s, h2agg_s):
    b = pl.program_id(0)
    i = pl.program_id(1)

    @pl.when(jnp.logical_and(b == 0, i == 0))
    def _():
        h1f = h1_ref[...].reshape(B * VSC, H)
        h2f = h2_ref[...].reshape(B * VST, H)
        uh1_s[...] = (jnp.dot(h1f, u1w_ref[...].T,
                              preferred_element_type=jnp.float32)
                      + u1b_ref[...]).reshape(B, VSC, H)
        uh2_s[...] = (jnp.dot(h2f, u2w_ref[...].T,
                              preferred_element_type=jnp.float32)
                      + u2b_ref[...]).reshape(B, VST, H)
        vh1_s[...] = (jnp.dot(h1f, v1w_ref[...].T,
                              preferred_element_type=jnp.float32)
                      + v1b_ref[...]).reshape(B, VSC, H)
        vh2_s[...] = (jnp.dot(h2f, v2w_ref[...].T,
                              preferred_element_type=jnp.float32)
                      + v2b_ref[...]).reshape(B, VST, H)

    mean = esum_ref[...] * (1.0 / N_EDGE)
    var = esumsq_ref[...] * (1.0 / N_EDGE) - mean * mean
    scale = jax.lax.rsqrt(var + EPS) * new_ref[...]
    shift = neb_ref[...] - mean * scale

    x = e_ref[0].reshape(BI2 * VST, H)
    ce = jnp.dot(x, cw_ref[...].T, preferred_element_type=jnp.float32)
    ah = ahf_ref[b, pl.ds(i * BI2, BI2), :]
    bh = bhf_ref[b]
    en = ce.reshape(BI2, VST, H) + ah[:, None, :] + bh[None, :, :]

    y = en.reshape(BI2 * VST, H) * scale + shift
    eout_ref[0] = e_ref[0] + jnp.maximum(y, 0.0).reshape(BI2, VST, H)

    g = jax.nn.sigmoid(en)
    h1agg_s[b, pl.ds(i * BI2, BI2), :] = jnp.sum(g * vh2_s[b][None, :, :],
                                                 axis=1)
    part2 = jnp.sum(g * vh1_s[b, pl.ds(i * BI2, BI2), :][:, None, :],
                    axis=0)

    @pl.when(i == 0)
    def _():
        h2agg_s[b] = part2

    @pl.when(i != 0)
    def _():
        h2agg_s[b] += part2

    @pl.when(jnp.logical_and(b == B - 1, i == NI2 - 1))
    def _():
        x1 = (uh1_s[...] + h1agg_s[...]).reshape(B * VSC, H)
        h1out_ref[...] = _bn_relu_res(
            x1, nhw_ref[...], nhb_ref[...],
            h1_ref[...].reshape(B * VSC, H)).reshape(B, VSC, H)
        x2 = (uh2_s[...] + h2agg_s[...]).reshape(B * VST, H)
        h2out_ref[...] = _bn_relu_res(
            x2, nhw_ref[...], nhb_ref[...],
            h2_ref[...].reshape(B * VST, H)).reshape(B, VST, H)


@functools.partial(jax.jit, static_argnames=())
def kernel(h1, h2, e, graph, U1_w, U1_b, V1_w, V1_b, U2_w, U2_b, V2_w, V2_b,
           A_w, A_b, B_w, B_b, C_w, C_b, nh_w, nh_b, ne_w, ne_b):
    del graph
    r = lambda v: v.reshape(1, H)

    full3 = lambda shape: pl.BlockSpec(shape, lambda b, i: (0, 0, 0))
    full2 = lambda shape: pl.BlockSpec(shape, lambda b, i: (0, 0))
    eblk1 = pl.BlockSpec((1, BI1, VST, H), lambda b, i: (b, i, 0, 0))
    eblk2 = pl.BlockSpec((1, BI2, VST, H), lambda b, i: (b, i, 0, 0))

    f32 = jnp.float32
    wspec = full2((H, H))
    bspec = full2((1, H))

    p1_out_shapes = (
        jax.ShapeDtypeStruct((1, H), f32),
        jax.ShapeDtypeStruct((1, H), f32),
        jax.ShapeDtypeStruct((B, VSC, H), f32),
        jax.ShapeDtypeStruct((B, VST, H), f32),
    )
    p1_out_specs = (
        bspec, bspec, full3((B, VSC, H)), full3((B, VST, H)),
    )
    p1_in_specs = (
        eblk1, full3((B, VSC, H)), full3((B, VST, H)),
        wspec, bspec, wspec, bspec, wspec, bspec,
    )
    esum, esumsq, ahf, bhf = pl.pallas_call(
        _pass1_body,
        grid=(B, NI1),
        in_specs=p1_in_specs,
        out_specs=p1_out_specs,
        out_shape=p1_out_shapes,
    )(e, h1, h2, A_w, r(A_b), B_w, r(B_b), C_w, r(C_b))

    p2_out_shapes = (
        jax.ShapeDtypeStruct((B, VSC, VST, H), f32),
        jax.ShapeDtypeStruct((B, VSC, H), f32),
        jax.ShapeDtypeStruct((B, VST, H), f32),
    )
    p2_out_specs = (eblk2, full3((B, VSC, H)), full3((B, VST, H)))
    p2_in_specs = (
        eblk2, full3((B, VSC, H)), full3((B, VST, H)), wspec,
        bspec, bspec,
        full3((B, VSC, H)), full3((B, VST, H)),
        wspec, bspec, wspec, bspec, wspec, bspec, wspec, bspec,
        bspec, bspec, bspec, bspec,
    )
    e_out, h1_out, h2_out = pl.pallas_call(
        _pass2_body,
        grid=(B, NI2),
        in_specs=p2_in_specs,
        out_specs=p2_out_specs,
        out_shape=p2_out_shapes,
        scratch_shapes=[
            pltpu.VMEM((B, VSC, H), f32),
            pltpu.VMEM((B, VST, H), f32),
            pltpu.VMEM((B, VSC, H), f32),
            pltpu.VMEM((B, VST, H), f32),
            pltpu.VMEM((B, VSC, H), f32),
            pltpu.VMEM((B, VST, H), f32),
        ],
    )(e, ahf, bhf, C_w, esum, esumsq, h1, h2,
      U1_w, r(U1_b), V1_w, r(V1_b), U2_w, r(U2_b), V2_w, r(V2_b),
      r(nh_w), r(nh_b), r(ne_w), r(ne_b))

    return (h1_out, h2_out, e_out)
